# baseline (speedup 1.0000x reference)
import functools

import jax
import jax.numpy as jnp
from jax import lax
from jax.experimental import pallas as pl
from jax.experimental.pallas import tpu as pltpu
from jax.experimental.pallas import tpu_sc as plsc

_SEQ = 8192
_W = 32
_DIM = _W * _W

_NBUF = 3
_CHUNK = 512


def _copy_body(table_hbm, out_hbm, *scratch):
    bufs = scratch[:_NBUF]
    sin = scratch[_NBUF:2 * _NBUF]
    sout = scratch[2 * _NBUF:]
    info = plsc.get_sparse_core_info()
    rows = _SEQ // info.num_cores
    nchunks = rows // _CHUNK
    base = lax.axis_index("c") * rows

    def in_copy(b, c):
        return pltpu.make_async_copy(
            table_hbm.at[pl.ds(base + c * _CHUNK, _CHUNK)], bufs[b], sin[b])

    def out_copy(b, c):
        return pltpu.make_async_copy(
            bufs[b], out_hbm.at[pl.ds(base + c * _CHUNK, _CHUNK)], sout[b])

    for b in range(_NBUF):
        in_copy(b, b).start()
    for c in range(nchunks):
        b = c % _NBUF
        in_copy(b, c).wait()
        out_copy(b, c).start()
        nxt = c + _NBUF
        if nxt < nchunks:
            out_copy(b, c).wait()
            in_copy(b, nxt).start()
    for c in range(max(0, nchunks - _NBUF), nchunks):
        out_copy(c % _NBUF, c).wait()


def kernel(x, position_embeddings):
    del x
    mesh = plsc.ScalarSubcoreMesh(axis_name="c", num_cores=2)
    copy = functools.partial(
        pl.kernel,
        mesh=mesh,
        out_type=jax.ShapeDtypeStruct((_SEQ, _DIM), jnp.float32),
        scratch_types=(
            [pltpu.VMEM_SHARED((_CHUNK, _DIM), jnp.float32)
             for _ in range(_NBUF)]
            + [pltpu.SemaphoreType.DMA for _ in range(2 * _NBUF)]
        ),
    )(_copy_body)
    out = copy(position_embeddings)
    return out.reshape(1, _SEQ, _W, _W)


# speedup vs baseline: 1.0197x; 1.0197x over previous
import jax
import jax.numpy as jnp
from jax import lax
from jax.experimental import pallas as pl
from jax.experimental.pallas import tpu as pltpu
from jax.experimental.pallas import tpu_sc as plsc
from jax._src.pallas import mpmd

_SEQ = 8192
_W = 32
_DIM = _W * _W

_SPLIT = 4096

_S_NBUF = 3
_S_CHUNK = 128

_T_NBUF = 3
_T_CHUNK = 32


def _ring_copy(table_hbm, out_hbm, bufs, sin, sout, base, rows, chunk):
    nbuf = len(bufs)
    nchunks = rows // chunk

    def in_copy(b, c):
        return pltpu.make_async_copy(
            table_hbm.at[pl.ds(base + c * chunk, chunk)], bufs[b], sin[b])

    def out_copy(b, c):
        return pltpu.make_async_copy(
            bufs[b], out_hbm.at[pl.ds(base + c * chunk, chunk)], sout[b])

    for b in range(min(nbuf, nchunks)):
        in_copy(b, b).start()
    for c in range(nchunks):
        b = c % nbuf
        in_copy(b, c).wait()
        out_copy(b, c).start()
        nxt = c + nbuf
        if nxt < nchunks:
            out_copy(b, c).wait()
            in_copy(b, nxt).start()
    for c in range(max(0, nchunks - nbuf), nchunks):
        out_copy(c % nbuf, c).wait()


def kernel(x, position_embeddings):
    del x
    scs_mesh = plsc.ScalarSubcoreMesh(axis_name="c", num_cores=2)
    tec_mesh = plsc.VectorSubcoreMesh(core_axis_name="c", subcore_axis_name="s")
    info = plsc.get_sparse_core_info()
    nc, ns = info.num_cores, info.num_subcores
    nw = nc * ns
    s_rows = _SPLIT // nc
    t_rows = (_SEQ - _SPLIT) // nw

    def scs_fn(table_hbm, out_hbm, *scratch):
        bufs = scratch[:_S_NBUF]
        sin = scratch[_S_NBUF:2 * _S_NBUF]
        sout = scratch[2 * _S_NBUF:3 * _S_NBUF]
        base = lax.axis_index("c") * s_rows
        _ring_copy(table_hbm, out_hbm, bufs, sin, sout, base, s_rows, _S_CHUNK)

    def tec_fn(table_hbm, out_hbm, *scratch):
        scratch = scratch[3 * _S_NBUF:]
        bufs = scratch[:_T_NBUF]
        sin = scratch[_T_NBUF:2 * _T_NBUF]
        sout = scratch[2 * _T_NBUF:3 * _T_NBUF]
        wid = lax.axis_index("s") * nc + lax.axis_index("c")
        base = _SPLIT + wid * t_rows
        _ring_copy(table_hbm, out_hbm, bufs, sin, sout, base, t_rows, _T_CHUNK)

    scratch_types = (
        [pltpu.VMEM_SHARED((_S_CHUNK, _DIM), jnp.float32)
         for _ in range(_S_NBUF)]
        + [pltpu.SemaphoreType.DMA @ scs_mesh for _ in range(2 * _S_NBUF)]
        + [(pltpu.VMEM @ tec_mesh)((_T_CHUNK, _DIM), jnp.float32)
           for _ in range(_T_NBUF)]
        + [pltpu.SemaphoreType.DMA @ tec_mesh for _ in range(2 * _T_NBUF)]
    )
    copy = mpmd.mpmd_map(
        [(scs_mesh, scs_fn), (tec_mesh, tec_fn)],
        out_types=[jax.ShapeDtypeStruct((_SEQ, _DIM), jnp.float32)],
        scratch_types=scratch_types,
    )
    (out,) = copy(position_embeddings)
    return out.reshape(1, _SEQ, _W, _W)
